# baseline (speedup 1.0000x reference)
import functools

import jax
import jax.numpy as jnp
from jax import lax
from jax.experimental import pallas as pl
from jax.experimental.pallas import tpu as pltpu
from jax.experimental.pallas import tpu_sc as plsc

N = 10000
E = 320000
D = 128
H = 8
C = 16
DH = 64
W = 80
NPAD = 10112
NT = 16
NCORE = 2
ROWS = NPAD // NT
EPAD = 331776
K = 128
NCHT = EPAD // K // NT
SEG = 18
NSEG = NCHT // SEG
NBUF = 3


def _pre_body(x_ref, lin_W_ref, asp_ref, p2_ref, hs2_ref, ad2_ref):
    h = jnp.dot(x_ref[...], lin_W_ref[...], preferred_element_type=jnp.float32)
    asp = jnp.dot(h, asp_ref[...], preferred_element_type=jnp.float32)
    hs2_ref[:NPAD, :DH] = h[:, :DH]
    hs2_ref[NPAD:, :DH] = h[:, DH:]
    hs2_ref[:NPAD, DH:] = asp
    hs2_ref[NPAD:, DH:] = asp
    ad2_ref[...] = jnp.dot(h, p2_ref[...], preferred_element_type=jnp.float32)


def _pre_call(x_pad, lin_W, ASP, P2):
    return pl.pallas_call(
        _pre_body,
        out_shape=(
            jax.ShapeDtypeStruct((2 * NPAD, W), jnp.float32),
            jax.ShapeDtypeStruct((NPAD, 2 * H), jnp.float32),
        ),
    )(x_pad, lin_W, ASP, P2)


def _lane_bcast(v, j):
    idx = jnp.full((16,), j, dtype=jnp.int32)
    return lax.gather(
        v, idx[:, None],
        lax.GatherDimensionNumbers(offset_dims=(), collapsed_slice_dims=(0,),
                                   start_index_map=(0,)),
        slice_sizes=(1,), mode=lax.GatherScatterMode.PROMISE_IN_BOUNDS)


def _edge_body(src2_hbm, dst2_hbm, hs2_hbm, ad2_hbm, zbig_hbm,
               outp_hbm,
               idxs0, idxd0, idxs1, idxd1,
               hs0, bd0, mx0, hs1, bd1, mx1,
               hs2b, bd2b, mx2b,
               out_sh,
               isem, gsem0, ssem0, gsem1, ssem1,
               gsem2, ssem2):
    c = lax.axis_index("c")
    s = lax.axis_index("s")
    r0 = s * ROWS

    pltpu.sync_copy(zbig_hbm.at[pl.ds(r0, ROWS)], out_sh.at[pl.ds(r0, ROWS)])
    plsc.subcore_barrier()

    rbase = s * NCHT
    islots = ((idxs0, idxd0), (idxs1, idxd1))

    def fire_refill(seg, slot):
        isl, idl = islots[slot]
        rows = pl.ds(rbase + seg * SEG, SEG)
        pltpu.async_copy(src2_hbm.at[c, rows], isl, isem)
        pltpu.async_copy(dst2_hbm.at[rows], idl, isem)

    def wait_refill(seg, slot):
        isl, idl = islots[slot]
        rows = pl.ds(rbase + seg * SEG, SEG)
        pltpu.make_async_copy(src2_hbm.at[c, rows], isl, isem).wait()
        pltpu.make_async_copy(dst2_hbm.at[rows], idl, isem).wait()

    sets = ((hs0, bd0, mx0, gsem0, ssem0),
            (hs1, bd1, mx1, gsem1, ssem1),
            (hs2b, bd2b, mx2b, gsem2, ssem2))

    def fire_gathers(i, S, slot):
        hs, bd2, mx, gsem, ssem = S
        isl, idl = islots[slot]
        pltpu.async_copy(hs2_hbm.at[isl.at[i]], hs, gsem)
        pltpu.async_copy(ad2_hbm.at[idl.at[i]], bd2, gsem)

    def wait_gathers(S):
        hs, bd2, mx, gsem, ssem = S
        pltpu.make_async_copy(hs2_hbm.at[idxs0.at[0]], hs, gsem).wait()
        pltpu.make_async_copy(ad2_hbm.at[idxd0.at[0]], bd2, gsem).wait()

    def fire_scatters(i, S, slot):
        hs, bd2, mx, gsem, ssem = S
        isl, idl = islots[slot]
        pltpu.async_copy(mx, out_sh.at[idl.at[i]], ssem, add=True)

    def wait_scatters(S):
        hs, bd2, mx, gsem, ssem = S
        pltpu.make_async_copy(mx, out_sh.at[idxd0.at[0]], ssem).wait()

    def compute(S):
        hs, bd2, mx, gsem, ssem = S

        @plsc.parallel_loop(0, K, 1, unroll=8)
        def edge(e):
            v = hs[e, pl.ds(DH, 16)] + bd2[e]
            ex = jnp.exp(jnp.maximum(v, 0.2 * v))
            mx[e, pl.ds(DH, 16)] = ex
            for j in range(DH // C):
                hv = hs[e, pl.ds(j * C, C)]
                mx[e, pl.ds(j * C, C)] = hv * _lane_bcast(ex, c * 4 + j)

    fire_refill(0, 0)
    wait_refill(0, 0)

    for seg in range(NSEG):
        slot = seg % 2

        if seg > 0:
            wait_refill(seg, slot)
        for b in range(NBUF):
            fire_gathers(b, sets[b], slot)
        if seg > 0:
            for b in range(NBUF):
                wait_scatters(sets[b])
        if seg + 1 < NSEG:
            fire_refill(seg + 1, 1 - slot)

        def pipe(t, carry, slot=slot, seg=seg):
            for b in range(NBUF):
                i = NBUF * t + b

                @pl.when(t > 0)
                def _(b=b):
                    wait_scatters(sets[b])
                wait_gathers(sets[b])
                compute(sets[b])
                fire_scatters(i, sets[b], slot)

                @pl.when(t < SEG // NBUF - 1)
                def _(i=i, b=b):
                    fire_gathers(i + NBUF, sets[b], slot)
            return carry

        lax.fori_loop(0, SEG // NBUF, pipe, 0)

    for b in range(NBUF):
        wait_scatters(sets[b])

    plsc.subcore_barrier()
    pltpu.sync_copy(out_sh.at[pl.ds(r0, ROWS)], outp_hbm.at[c, pl.ds(r0, ROWS)])


@functools.partial(
    pl.kernel,
    out_type=jax.ShapeDtypeStruct((NCORE, NPAD, W), jnp.float32),
    mesh=plsc.VectorSubcoreMesh(core_axis_name="c", subcore_axis_name="s"),
    compiler_params=pltpu.CompilerParams(use_tc_tiling_on_sc=False),
    scratch_types=[
        pltpu.VMEM((SEG, K), jnp.int32),
        pltpu.VMEM((SEG, K), jnp.int32),
        pltpu.VMEM((SEG, K), jnp.int32),
        pltpu.VMEM((SEG, K), jnp.int32),
        pltpu.VMEM((K, W), jnp.float32),
        pltpu.VMEM((K, 2 * H), jnp.float32),
        pltpu.VMEM((K, W), jnp.float32),
        pltpu.VMEM((K, W), jnp.float32),
        pltpu.VMEM((K, 2 * H), jnp.float32),
        pltpu.VMEM((K, W), jnp.float32),
        pltpu.VMEM((K, W), jnp.float32),
        pltpu.VMEM((K, 2 * H), jnp.float32),
        pltpu.VMEM((K, W), jnp.float32),
        pltpu.VMEM_SHARED((NPAD, W), jnp.float32),
        pltpu.SemaphoreType.DMA,
        pltpu.SemaphoreType.DMA,
        pltpu.SemaphoreType.DMA,
        pltpu.SemaphoreType.DMA,
        pltpu.SemaphoreType.DMA,
        pltpu.SemaphoreType.DMA,
        pltpu.SemaphoreType.DMA,
    ],
)
def _edge_call(src2_hbm, dst2_hbm, hs2_hbm, ad2_hbm, zbig_hbm,
               outp_hbm, *scratch):
    _edge_body(src2_hbm, dst2_hbm, hs2_hbm, ad2_hbm, zbig_hbm,
               outp_hbm, *scratch)


def _elu(v):
    return jnp.where(v > 0, v, jnp.exp(v) - 1.0)


def _bn(v, g, b):
    mu = jnp.mean(v, axis=0, keepdims=True)
    var = jnp.mean((v - mu) * (v - mu), axis=0, keepdims=True)
    return (v - mu) * jax.lax.rsqrt(var + 1e-5) * g + b


def _post_body(outp_ref, x_ref, res_W_ref, conv_bias_ref,
               norm_g_ref, norm_b_ref, down_W_ref, down_b_ref,
               bn1_g_ref, bn1_b_ref, up_W_ref, up_b_ref,
               bn2_g_ref, bn2_b_ref, emask_ref, out_ref):
    agg = jnp.concatenate([outp_ref[0, :N, :DH], outp_ref[1, :N, :DH]], axis=1)
    es = outp_ref[0, :N, DH:]
    recip = 1.0 / (es + 1e-16)
    den_big = jnp.dot(recip, emask_ref[...],
                      preferred_element_type=jnp.float32)
    x = x_ref[...]
    gat = agg * den_big + jnp.dot(x, res_W_ref[...],
                                  preferred_element_type=jnp.float32)
    gat = gat + conv_bias_ref[...]
    gat = _elu(_bn(gat, norm_g_ref[...], norm_b_ref[...]))
    z = jnp.dot(gat, down_W_ref[...], preferred_element_type=jnp.float32)
    z = _elu(_bn(z + down_b_ref[...], bn1_g_ref[...], bn1_b_ref[...]))
    z = jnp.dot(z, up_W_ref[...], preferred_element_type=jnp.float32)
    z = _elu(_bn(z + up_b_ref[...], bn2_g_ref[...], bn2_b_ref[...]))
    out_ref[...] = gat + z + x


def _post_call(outp, x, res_W, conv_bias, norm_g, norm_b, down_W,
               down_b, bn1_g, bn1_b, up_W, up_b, bn2_g, bn2_b, emask):
    return pl.pallas_call(
        _post_body,
        out_shape=jax.ShapeDtypeStruct((N, D), jnp.float32),
    )(outp, x, res_W, conv_bias, norm_g, norm_b, down_W, down_b,
      bn1_g, bn1_b, up_W, up_b, bn2_g, bn2_b, emask)


def kernel(x, edge_index, lin_W, att_src, att_dst, conv_bias, res_W,
           norm_g, norm_b, down_W, down_b, bn1_g, bn1_b, up_W, up_b,
           bn2_g, bn2_b):
    f32 = jnp.float32
    x_pad = jnp.pad(x, ((0, NPAD - N), (0, 0)))

    hc = jnp.arange(D, dtype=jnp.int32) // C
    heads = jnp.arange(H, dtype=jnp.int32)
    M = (hc[:, None] == heads[None, :]).astype(f32)
    A_s = att_src.reshape(-1)[:, None] * M
    A_d = att_dst.reshape(-1)[:, None] * M
    ASP = jnp.concatenate([A_s, jnp.zeros((D, H), f32)], axis=1)
    P2 = jnp.concatenate([A_d, A_s], axis=1)
    emask = jnp.concatenate([M.T, jnp.zeros((H, D), f32)], axis=0)

    hs2, ad2 = _pre_call(x_pad, lin_W, ASP, P2)

    fill = jnp.full((EPAD - E,), N, jnp.int32)
    src = jnp.concatenate([edge_index[0], fill])
    src2 = jnp.stack([src, src + NPAD]).reshape(NCORE, EPAD // K, K)
    dst = jnp.concatenate([edge_index[1], fill]).reshape(EPAD // K, K)

    zbig = jnp.zeros((NPAD, W), f32)
    outp = _edge_call(src2, dst, hs2, ad2, zbig)

    out = _post_call(
        outp, x, res_W,
        conv_bias.reshape(1, D), norm_g.reshape(1, D), norm_b.reshape(1, D),
        down_W, down_b.reshape(1, -1), bn1_g.reshape(1, -1),
        bn1_b.reshape(1, -1), up_W, up_b.reshape(1, D),
        bn2_g.reshape(1, D), bn2_b.reshape(1, D), emask)
    return out


# speedup vs baseline: 1.2495x; 1.2495x over previous
import functools

import jax
import jax.numpy as jnp
from jax import lax
from jax.experimental import pallas as pl
from jax.experimental.pallas import tpu as pltpu
from jax.experimental.pallas import tpu_sc as plsc

N = 10000
E = 320000
D = 128
H = 8
C = 16
DH = 64
W = 80
NPAD = 10112
NT = 16
NCORE = 2
ROWS = NPAD // NT
EPAD = 327680
K = 128
NCHT = EPAD // K // NT
SEG = 20
NSEG = NCHT // SEG
NBUF = 2


def _pre_body(x_ref, lin_W_ref, asp_ref, p2_ref, hs2_ref, ad2_ref):
    h = jnp.dot(x_ref[...], lin_W_ref[...], preferred_element_type=jnp.float32)
    asp = jnp.dot(h, asp_ref[...], preferred_element_type=jnp.float32)
    hs2_ref[:NPAD, :DH] = h[:, :DH]
    hs2_ref[NPAD:, :DH] = h[:, DH:]
    hs2_ref[:NPAD, DH:] = asp
    hs2_ref[NPAD:, DH:] = asp
    ad2_ref[...] = jnp.dot(h, p2_ref[...], preferred_element_type=jnp.float32)


def _pre_call(x_pad, lin_W, ASP, P2):
    return pl.pallas_call(
        _pre_body,
        out_shape=(
            jax.ShapeDtypeStruct((2 * NPAD, W), jnp.float32),
            jax.ShapeDtypeStruct((NPAD, 2 * H), jnp.float32),
        ),
    )(x_pad, lin_W, ASP, P2)


def _lane_bcast(v, j):
    idx = jnp.full((16,), j, dtype=jnp.int32)
    return lax.gather(
        v, idx[:, None],
        lax.GatherDimensionNumbers(offset_dims=(), collapsed_slice_dims=(0,),
                                   start_index_map=(0,)),
        slice_sizes=(1,), mode=lax.GatherScatterMode.PROMISE_IN_BOUNDS)


def _edge_body(src2_hbm, dst2_hbm, hs2_hbm, ad2_hbm, zbig_hbm,
               outp_hbm,
               idxs0, idxd0, idxs1, idxd1,
               hs0, bd0, mx0, hs1, bd1, mx1,
               out_sh,
               isem, gsem0, ssem0, gsem1, ssem1):
    c = lax.axis_index("c")
    s = lax.axis_index("s")
    r0 = s * ROWS

    pltpu.sync_copy(zbig_hbm.at[pl.ds(r0, ROWS)], out_sh.at[pl.ds(r0, ROWS)])
    plsc.subcore_barrier()

    rbase = s * NCHT
    islots = ((idxs0, idxd0), (idxs1, idxd1))

    def fire_refill(seg, slot):
        isl, idl = islots[slot]
        rows = pl.ds(rbase + seg * SEG, SEG)
        pltpu.async_copy(src2_hbm.at[c, rows], isl, isem)
        pltpu.async_copy(dst2_hbm.at[rows], idl, isem)

    def wait_refill(seg, slot):
        isl, idl = islots[slot]
        rows = pl.ds(rbase + seg * SEG, SEG)
        pltpu.make_async_copy(src2_hbm.at[c, rows], isl, isem).wait()
        pltpu.make_async_copy(dst2_hbm.at[rows], idl, isem).wait()

    sets = ((hs0, bd0, mx0, gsem0, ssem0),
            (hs1, bd1, mx1, gsem1, ssem1))

    def fire_gathers(i, S, slot):
        hs, bd2, mx, gsem, ssem = S
        isl, idl = islots[slot]
        pltpu.async_copy(hs2_hbm.at[isl.at[i]], hs, gsem)
        pltpu.async_copy(ad2_hbm.at[idl.at[i]], bd2, gsem)

    def wait_gathers(S):
        hs, bd2, mx, gsem, ssem = S
        pltpu.make_async_copy(hs2_hbm.at[idxs0.at[0]], hs, gsem).wait()
        pltpu.make_async_copy(ad2_hbm.at[idxd0.at[0]], bd2, gsem).wait()

    def fire_scatters(i, S, slot):
        hs, bd2, mx, gsem, ssem = S
        isl, idl = islots[slot]
        pltpu.async_copy(mx, out_sh.at[idl.at[i]], ssem, add=True)

    def wait_scatters(S):
        hs, bd2, mx, gsem, ssem = S
        pltpu.make_async_copy(mx, out_sh.at[idxd0.at[0]], ssem).wait()

    def compute(S):
        hs, bd2, mx, gsem, ssem = S

        @plsc.parallel_loop(0, K, 1, unroll=8)
        def edge(e):
            v = hs[e, pl.ds(DH, 16)] + bd2[e]
            ex = jnp.exp(jnp.maximum(v, 0.2 * v))
            mx[e, pl.ds(DH, 16)] = ex
            for j in range(DH // C):
                hv = hs[e, pl.ds(j * C, C)]
                mx[e, pl.ds(j * C, C)] = hv * _lane_bcast(ex, c * 4 + j)

    fire_refill(0, 0)
    wait_refill(0, 0)

    for seg in range(NSEG):
        slot = seg % 2

        if seg > 0:
            wait_refill(seg, slot)
        for b in range(NBUF):
            fire_gathers(b, sets[b], slot)
        if seg > 0:
            for b in range(NBUF):
                wait_scatters(sets[b])
        if seg + 1 < NSEG:
            fire_refill(seg + 1, 1 - slot)

        def pipe(t, carry, slot=slot, seg=seg):
            for b in range(NBUF):
                i = NBUF * t + b

                @pl.when(t > 0)
                def _(b=b):
                    wait_scatters(sets[b])
                wait_gathers(sets[b])
                compute(sets[b])
                fire_scatters(i, sets[b], slot)

                @pl.when(t < SEG // NBUF - 1)
                def _(i=i, b=b):
                    fire_gathers(i + NBUF, sets[b], slot)
            return carry

        lax.fori_loop(0, SEG // NBUF, pipe, 0)

    for b in range(NBUF):
        wait_scatters(sets[b])

    plsc.subcore_barrier()
    pltpu.sync_copy(out_sh.at[pl.ds(r0, ROWS)], outp_hbm.at[c, pl.ds(r0, ROWS)])


@functools.partial(
    pl.kernel,
    out_type=jax.ShapeDtypeStruct((NCORE, NPAD, W), jnp.float32),
    mesh=plsc.VectorSubcoreMesh(core_axis_name="c", subcore_axis_name="s"),
    compiler_params=pltpu.CompilerParams(use_tc_tiling_on_sc=False),
    scratch_types=[
        pltpu.VMEM((SEG, K), jnp.int32),
        pltpu.VMEM((SEG, K), jnp.int32),
        pltpu.VMEM((SEG, K), jnp.int32),
        pltpu.VMEM((SEG, K), jnp.int32),
        pltpu.VMEM((K, W), jnp.float32),
        pltpu.VMEM((K, 2 * H), jnp.float32),
        pltpu.VMEM((K, W), jnp.float32),
        pltpu.VMEM((K, W), jnp.float32),
        pltpu.VMEM((K, 2 * H), jnp.float32),
        pltpu.VMEM((K, W), jnp.float32),
        pltpu.VMEM_SHARED((NPAD, W), jnp.float32),
        pltpu.SemaphoreType.DMA,
        pltpu.SemaphoreType.DMA,
        pltpu.SemaphoreType.DMA,
        pltpu.SemaphoreType.DMA,
        pltpu.SemaphoreType.DMA,
    ],
)
def _edge_call(src2_hbm, dst2_hbm, hs2_hbm, ad2_hbm, zbig_hbm,
               outp_hbm, *scratch):
    _edge_body(src2_hbm, dst2_hbm, hs2_hbm, ad2_hbm, zbig_hbm,
               outp_hbm, *scratch)


def _elu(v):
    return jnp.where(v > 0, v, jnp.exp(v) - 1.0)


def _bn(v, g, b):
    mu = jnp.mean(v, axis=0, keepdims=True)
    var = jnp.mean((v - mu) * (v - mu), axis=0, keepdims=True)
    return (v - mu) * jax.lax.rsqrt(var + 1e-5) * g + b


def _post_body(outp_ref, x_ref, res_W_ref, conv_bias_ref,
               norm_g_ref, norm_b_ref, down_W_ref, down_b_ref,
               bn1_g_ref, bn1_b_ref, up_W_ref, up_b_ref,
               bn2_g_ref, bn2_b_ref, emask_ref, out_ref):
    agg = jnp.concatenate([outp_ref[0, :N, :DH], outp_ref[1, :N, :DH]], axis=1)
    es = outp_ref[0, :N, DH:]
    recip = 1.0 / (es + 1e-16)
    den_big = jnp.dot(recip, emask_ref[...],
                      preferred_element_type=jnp.float32)
    x = x_ref[...]
    gat = agg * den_big + jnp.dot(x, res_W_ref[...],
                                  preferred_element_type=jnp.float32)
    gat = gat + conv_bias_ref[...]
    gat = _elu(_bn(gat, norm_g_ref[...], norm_b_ref[...]))
    z = jnp.dot(gat, down_W_ref[...], preferred_element_type=jnp.float32)
    z = _elu(_bn(z + down_b_ref[...], bn1_g_ref[...], bn1_b_ref[...]))
    z = jnp.dot(z, up_W_ref[...], preferred_element_type=jnp.float32)
    z = _elu(_bn(z + up_b_ref[...], bn2_g_ref[...], bn2_b_ref[...]))
    out_ref[...] = gat + z + x


def _post_call(outp, x, res_W, conv_bias, norm_g, norm_b, down_W,
               down_b, bn1_g, bn1_b, up_W, up_b, bn2_g, bn2_b, emask):
    return pl.pallas_call(
        _post_body,
        out_shape=jax.ShapeDtypeStruct((N, D), jnp.float32),
    )(outp, x, res_W, conv_bias, norm_g, norm_b, down_W, down_b,
      bn1_g, bn1_b, up_W, up_b, bn2_g, bn2_b, emask)


def kernel(x, edge_index, lin_W, att_src, att_dst, conv_bias, res_W,
           norm_g, norm_b, down_W, down_b, bn1_g, bn1_b, up_W, up_b,
           bn2_g, bn2_b):
    f32 = jnp.float32
    x_pad = jnp.pad(x, ((0, NPAD - N), (0, 0)))

    hc = jnp.arange(D, dtype=jnp.int32) // C
    heads = jnp.arange(H, dtype=jnp.int32)
    M = (hc[:, None] == heads[None, :]).astype(f32)
    A_s = att_src.reshape(-1)[:, None] * M
    A_d = att_dst.reshape(-1)[:, None] * M
    ASP = jnp.concatenate([A_s, jnp.zeros((D, H), f32)], axis=1)
    P2 = jnp.concatenate([A_d, A_s], axis=1)
    emask = jnp.concatenate([M.T, jnp.zeros((H, D), f32)], axis=0)

    hs2, ad2 = _pre_call(x_pad, lin_W, ASP, P2)

    fill = jnp.full((EPAD - E,), N, jnp.int32)
    src = jnp.concatenate([edge_index[0], fill])
    src2 = jnp.stack([src, src + NPAD]).reshape(NCORE, EPAD // K, K)
    dst = jnp.concatenate([edge_index[1], fill]).reshape(EPAD // K, K)

    zbig = jnp.zeros((NPAD, W), f32)
    outp = _edge_call(src2, dst, hs2, ad2, zbig)

    out = _post_call(
        outp, x, res_W,
        conv_bias.reshape(1, D), norm_g.reshape(1, D), norm_b.reshape(1, D),
        down_W, down_b.reshape(1, -1), bn1_g.reshape(1, -1),
        bn1_b.reshape(1, -1), up_W, up_b.reshape(1, D),
        bn2_g.reshape(1, D), bn2_b.reshape(1, D), emask)
    return out
